# baseline (speedup 1.0000x reference)
import functools

import jax
import jax.numpy as jnp
from jax.experimental import pallas as pl
from jax.experimental.pallas import tpu as pltpu


def _conv_stats_kernel(x_ref, w_ref, y_ref, stats_ref, *, kh, kw, h, wd, cin,
                       pad_lanes):
    hw = h * wd
    xb = x_ref[...].astype(jnp.bfloat16)
    zpad = jnp.zeros((cin, pad_lanes), jnp.bfloat16)
    flat = jnp.concatenate([zpad, xb, zpad], axis=1)

    col = jax.lax.broadcasted_iota(jnp.int32, (1, hw), 1) % wd
    zero_b = jnp.zeros((cin, hw), jnp.bfloat16)

    parts = []
    for i in range(kh):
        di = i - (kh - 1) // 2
        for j in range(kw):
            dj = j - (kw - 1) // 2
            start = pad_lanes + di * wd + dj
            sl = jax.lax.slice(flat, (0, start), (cin, start + hw))
            if dj < 0:
                sl = jnp.where(col < -dj, zero_b, sl)
            elif dj > 0:
                sl = jnp.where(col >= wd - dj, zero_b, sl)
            parts.append(sl)
    rhs = jnp.concatenate(parts, axis=0)

    acc = jnp.dot(w_ref[...], rhs,
                  preferred_element_type=jnp.float32)
    y_ref[...] = acc.astype(y_ref.dtype)
    stats_ref[:, 0:1] = jnp.sum(acc, axis=1, keepdims=True)
    stats_ref[:, 1:2] = jnp.sum(acc * acc, axis=1, keepdims=True)


def _bn_hsw_kernel(y_ref, scale_ref, shift_ref, out_ref):
    z = y_ref[...].astype(jnp.float32) * scale_ref[...] + shift_ref[...]
    out_ref[...] = z * jnp.clip(z + 3.0, 0.0, 6.0) * (1.0 / 6.0)


def kernel(x_nchw, w, gamma, beta):
    n, cin, h, wd = x_nchw.shape
    cout, cin_w, kh, kw = w.shape
    assert cin_w == cin
    hw = h * wd
    kk = kh * kw
    pad_lanes = wd + 8

    x_flat = x_nchw.reshape(n, cin, hw)
    wt = jnp.transpose(w, (0, 2, 3, 1)).reshape(cout, kk * cin)
    wt = wt.astype(jnp.bfloat16)

    vmem_limit = 56 * 1024 * 1024

    y, stats = pl.pallas_call(
        functools.partial(_conv_stats_kernel, kh=kh, kw=kw, h=h, wd=wd,
                          cin=cin, pad_lanes=pad_lanes),
        out_shape=(jax.ShapeDtypeStruct((n, cout, hw), jnp.bfloat16),
                   jax.ShapeDtypeStruct((n, cout, 2), jnp.float32)),
        grid=(n,),
        in_specs=[pl.BlockSpec((None, cin, hw), lambda b: (b, 0, 0)),
                  pl.BlockSpec((cout, kk * cin), lambda b: (0, 0))],
        out_specs=(pl.BlockSpec((None, cout, hw), lambda b: (b, 0, 0)),
                   pl.BlockSpec((None, cout, 2), lambda b: (b, 0, 0))),
        compiler_params=pltpu.CompilerParams(
            dimension_semantics=("parallel",),
            vmem_limit_bytes=vmem_limit),
        cost_estimate=pl.CostEstimate(
            flops=2 * n * hw * kk * cin * cout,
            transcendentals=0,
            bytes_accessed=(n * cin * hw * 4 + cout * kk * cin * 2
                            + n * cout * hw * 2 + n * cout * 2 * 4)),
    )(x_flat, wt)

    m_real = float(n * hw)
    ssum = jnp.sum(stats, axis=0)
    mean = ssum[:, 0] * (1.0 / m_real)
    var = jnp.maximum(ssum[:, 1] * (1.0 / m_real) - mean * mean, 0.0)
    inv_std = jax.lax.rsqrt(var + 1e-5)
    g = gamma.astype(jnp.float32)
    scale = (g * inv_std).reshape(cout, 1)
    shift = (beta.astype(jnp.float32) - mean * g * inv_std).reshape(cout, 1)

    out = pl.pallas_call(
        _bn_hsw_kernel,
        out_shape=jax.ShapeDtypeStruct((n, cout, hw), jnp.float32),
        grid=(n,),
        in_specs=[pl.BlockSpec((None, cout, hw), lambda b: (b, 0, 0)),
                  pl.BlockSpec((cout, 1), lambda b: (0, 0)),
                  pl.BlockSpec((cout, 1), lambda b: (0, 0))],
        out_specs=pl.BlockSpec((None, cout, hw), lambda b: (b, 0, 0)),
        compiler_params=pltpu.CompilerParams(
            dimension_semantics=("parallel",),
            vmem_limit_bytes=vmem_limit),
        cost_estimate=pl.CostEstimate(
            flops=8 * n * cout * hw,
            transcendentals=0,
            bytes_accessed=n * cout * hw * 6 + cout * 8),
    )(y, scale, shift)

    return out.reshape(n, cout, h, wd)
